# baseline (speedup 1.0000x reference)
import functools

import jax
import jax.numpy as jnp
from jax import lax
from jax.experimental import pallas as pl
from jax.experimental.pallas import tpu as pltpu
from jax.experimental.pallas import tpu_sc as plsc

NC = 2
NS = 16
NW = NC * NS
CHUNK = 128


def _seg_sum_call(n, d, k0, k1):
    rows_per_sub = (n // NS) // 8 * 8
    tail = n - rows_per_sub * NS
    n_pad = n + 8
    mesh = plsc.VectorSubcoreMesh(
        core_axis_name="c", subcore_axis_name="s", num_cores=NC, num_subcores=NS
    )

    assert k0 % 2 == 0 and k1 % 2 == 0 and k0 >= 2 and k1 >= 2
    k_max = max(k0, k1)

    @functools.partial(
        pl.kernel,
        out_type=jax.ShapeDtypeStruct((NC, n, d), jnp.float32),
        mesh=mesh,
        scratch_types=[
            pltpu.VMEM_SHARED((n_pad, d), jnp.float32),
            [pltpu.VMEM((1, CHUNK), jnp.int32) for _ in range(2)],
            [pltpu.VMEM((1, CHUNK), jnp.int32) for _ in range(2)],
            [pltpu.VMEM((CHUNK, d), jnp.float32) for _ in range(2)],
            [pltpu.SemaphoreType.DMA for _ in range(2)],
            [pltpu.SemaphoreType.DMA for _ in range(2)],
        ],
    )
    def seg(x_hbm, src_hbm, dst_hbm, out_hbm, acc, sidx, didx, rows, gsem,
            isem):
        c = lax.axis_index("c")
        s = lax.axis_index("s")
        nc = jnp.where(c == 0, k0, k1)
        r0 = s * rows_per_sub
        pltpu.sync_copy(
            x_hbm.at[pl.ds(r0, rows_per_sub)], acc.at[pl.ds(r0, rows_per_sub)]
        )
        if tail:
            @pl.when(s == NS - 1)
            def _():
                t0 = rows_per_sub * NS
                pltpu.sync_copy(
                    x_hbm.at[pl.ds(t0, tail)], acc.at[pl.ds(t0, tail)]
                )
        plsc.subcore_barrier()
        base = jnp.where(c == 0, s * k0, NS * k0 + s * k1)

        def load_idx(k, b):
            pltpu.async_copy(src_hbm.at[k], sidx[b], isem[b])
            pltpu.async_copy(dst_hbm.at[k], didx[b], isem[b])

        def wait_idx(b):
            pltpu.make_async_copy(dst_hbm.at[0], didx[b], isem[b]).wait()
            pltpu.make_async_copy(src_hbm.at[0], sidx[b], isem[b]).wait()

        load_idx(base, 0)
        wait_idx(0)
        pltpu.async_copy(x_hbm.at[sidx[0].at[0]], rows[0], gsem[0])
        load_idx(base + 1, 1)

        def step(k, b):
            @pl.when(k + 1 < nc)
            def _():
                wait_idx(1 - b)
                pltpu.async_copy(x_hbm.at[sidx[1 - b].at[0]], rows[1 - b],
                                 gsem[1 - b])

            @pl.when(k < nc)
            def _():
                pltpu.make_async_copy(
                    x_hbm.at[sidx[b].at[0]], rows[b], gsem[b]).wait()
                pltpu.sync_copy(rows[b], acc.at[didx[b].at[0]], add=True)

            @pl.when(k + 2 < nc)
            def _():
                load_idx(base + k + 2, b)

        def body(j, carry):
            step(j * 2, 0)
            step(j * 2 + 1, 1)
            return carry

        lax.fori_loop(0, k_max // 2, body, 0)
        plsc.subcore_barrier()
        pltpu.sync_copy(
            acc.at[pl.ds(r0, rows_per_sub)],
            out_hbm.at[c].at[pl.ds(r0, rows_per_sub)],
        )
        if tail:
            @pl.when(s == NS - 1)
            def _():
                t0 = rows_per_sub * NS
                pltpu.sync_copy(
                    acc.at[pl.ds(t0, tail)], out_hbm.at[c].at[pl.ds(t0, tail)]
                )

    return seg


def _mlp_body(part, xref, w1, b1, g1, be1, w2, b2, g2, be2, out, *, last_relu):
    h = part[0] + part[1] - xref[...]
    h = lax.dot_general(
        h, w1[...], (((1,), (1,)), ((), ())),
        preferred_element_type=jnp.float32,
    ) + b1[...]
    m = jnp.mean(h, axis=0, keepdims=True)
    v = jnp.mean((h - m) * (h - m), axis=0, keepdims=True)
    h = g1[...] * (h - m) / jnp.sqrt(v + 1e-5) + be1[...]
    h = jnp.maximum(h, 0.0)
    h = lax.dot_general(
        h, w2[...], (((1,), (1,)), ((), ())),
        preferred_element_type=jnp.float32,
    ) + b2[...]
    m = jnp.mean(h, axis=0, keepdims=True)
    v = jnp.mean((h - m) * (h - m), axis=0, keepdims=True)
    h = g2[...] * (h - m) / jnp.sqrt(v + 1e-5) + be2[...]
    if last_relu:
        h = jnp.maximum(h, 0.0)
    out[...] = h


def _mlp_call(n, d_out, last_relu):
    return pl.pallas_call(
        functools.partial(_mlp_body, last_relu=last_relu),
        out_shape=jax.ShapeDtypeStruct((n, d_out), jnp.float32),
    )


def kernel(x, edge_index, W1a, b1a, g1a, be1a, W1b, b1b, g1b, be1b,
           W2a, b2a, g2a, be2a, W2b, b2b, g2b, be2b):
    n, d = x.shape
    e = edge_index.shape[1]
    h_dim = W1a.shape[0]
    c_dim = W2b.shape[0]

    t_chunks = -(-e // (NS * CHUNK))
    t_chunks += t_chunks % 2
    k0, k1 = 108, 50
    assert NS * (k0 + k1) * CHUNK >= e
    e_pad = NS * (k0 + k1) * CHUNK
    src = edge_index[0]
    dst = edge_index[1]
    if e_pad != e:
        pad = e_pad - e
        src = jnp.concatenate([src, jnp.zeros((pad,), jnp.int32)])
        dst = jnp.concatenate([dst, jnp.full((pad,), n, jnp.int32)])
    src = src.reshape(-1, 1, CHUNK)
    dst = dst.reshape(-1, 1, CHUNK)

    seg = _seg_sum_call(n, d, k0, k1)
    r = lambda a: a.reshape(1, -1)

    part1 = seg(x, src, dst)
    h = _mlp_call(n, h_dim, True)(
        part1, x, W1a, r(b1a), r(g1a), r(be1a), W1b, r(b1b), r(g1b), r(be1b)
    )
    part2 = seg(h, src, dst)
    out = _mlp_call(n, c_dim, False)(
        part2, h, W2a, r(b2a), r(g2a), r(be2a), W2b, r(b2b), r(g2b), r(be2b)
    )
    return out


# speedup vs baseline: 1.0587x; 1.0587x over previous
import functools

import jax
import jax.numpy as jnp
from jax import lax
from jax.experimental import pallas as pl
from jax.experimental.pallas import tpu as pltpu
from jax.experimental.pallas import tpu_sc as plsc

NC = 2
NS = 16
NW = NC * NS
CHUNK = 128


def _seg_sum_call(n, d, k0, k1):
    rows_per_sub = (n // NS) // 8 * 8
    tail = n - rows_per_sub * NS
    n_pad = n + 8
    mesh = plsc.VectorSubcoreMesh(
        core_axis_name="c", subcore_axis_name="s", num_cores=NC, num_subcores=NS
    )

    assert k0 % 2 == 0 and k1 % 2 == 0 and k0 >= 2 and k1 >= 2
    k_max = max(k0, k1)

    @functools.partial(
        pl.kernel,
        out_type=jax.ShapeDtypeStruct((NC, n, d), jnp.float32),
        mesh=mesh,
        scratch_types=[
            pltpu.VMEM_SHARED((n_pad, d), jnp.float32),
            [pltpu.VMEM((1, CHUNK), jnp.int32) for _ in range(2)],
            [pltpu.VMEM((1, CHUNK), jnp.int32) for _ in range(2)],
            [pltpu.VMEM((CHUNK, d), jnp.float32) for _ in range(2)],
            [pltpu.SemaphoreType.DMA for _ in range(2)],
            [pltpu.SemaphoreType.DMA for _ in range(2)],
        ],
    )
    def seg(x_hbm, src_hbm, dst_hbm, out_hbm, acc, sidx, didx, rows, gsem,
            isem):
        c = lax.axis_index("c")
        s = lax.axis_index("s")
        nc = jnp.where(c == 0, k0, k1)
        r0 = s * rows_per_sub
        pltpu.sync_copy(
            x_hbm.at[pl.ds(r0, rows_per_sub)], acc.at[pl.ds(r0, rows_per_sub)]
        )
        if tail:
            @pl.when(s == NS - 1)
            def _():
                t0 = rows_per_sub * NS
                pltpu.sync_copy(
                    x_hbm.at[pl.ds(t0, tail)], acc.at[pl.ds(t0, tail)]
                )
        plsc.subcore_barrier()
        base = jnp.where(c == 0, s * k0, NS * k0 + s * k1)

        def load_idx(k, b):
            pltpu.async_copy(src_hbm.at[k], sidx[b], isem[b])
            pltpu.async_copy(dst_hbm.at[k], didx[b], isem[b])

        def wait_idx(b):
            pltpu.make_async_copy(dst_hbm.at[0], didx[b], isem[b]).wait()
            pltpu.make_async_copy(src_hbm.at[0], sidx[b], isem[b]).wait()

        load_idx(base, 0)
        wait_idx(0)
        pltpu.async_copy(x_hbm.at[sidx[0].at[0]], rows[0], gsem[0])
        load_idx(base + 1, 1)

        def step(k, b):
            @pl.when(k + 1 < nc)
            def _():
                wait_idx(1 - b)
                pltpu.async_copy(x_hbm.at[sidx[1 - b].at[0]], rows[1 - b],
                                 gsem[1 - b])

            @pl.when(k < nc)
            def _():
                pltpu.make_async_copy(
                    x_hbm.at[sidx[b].at[0]], rows[b], gsem[b]).wait()
                pltpu.sync_copy(rows[b], acc.at[didx[b].at[0]], add=True)

            @pl.when(k + 2 < nc)
            def _():
                load_idx(base + k + 2, b)

        def body(j, carry):
            step(j * 2, 0)
            step(j * 2 + 1, 1)
            return carry

        lax.fori_loop(0, k_max // 2, body, 0)
        plsc.subcore_barrier()
        pltpu.sync_copy(
            acc.at[pl.ds(r0, rows_per_sub)],
            out_hbm.at[c].at[pl.ds(r0, rows_per_sub)],
        )
        if tail:
            @pl.when(s == NS - 1)
            def _():
                t0 = rows_per_sub * NS
                pltpu.sync_copy(
                    acc.at[pl.ds(t0, tail)], out_hbm.at[c].at[pl.ds(t0, tail)]
                )

    return seg


def _mlp_body(part, xref, w1, b1, g1, be1, w2, b2, g2, be2, out, *, last_relu):
    h = part[0] + part[1] - xref[...]
    h = lax.dot_general(
        h, w1[...], (((1,), (1,)), ((), ())),
        preferred_element_type=jnp.float32,
    ) + b1[...]
    m = jnp.mean(h, axis=0, keepdims=True)
    v = jnp.mean((h - m) * (h - m), axis=0, keepdims=True)
    h = g1[...] * (h - m) / jnp.sqrt(v + 1e-5) + be1[...]
    h = jnp.maximum(h, 0.0)
    h = lax.dot_general(
        h, w2[...], (((1,), (1,)), ((), ())),
        preferred_element_type=jnp.float32,
    ) + b2[...]
    m = jnp.mean(h, axis=0, keepdims=True)
    v = jnp.mean((h - m) * (h - m), axis=0, keepdims=True)
    h = g2[...] * (h - m) / jnp.sqrt(v + 1e-5) + be2[...]
    if last_relu:
        h = jnp.maximum(h, 0.0)
    out[...] = h


def _mlp_call(n, d_out, last_relu):
    return pl.pallas_call(
        functools.partial(_mlp_body, last_relu=last_relu),
        out_shape=jax.ShapeDtypeStruct((n, d_out), jnp.float32),
    )


def kernel(x, edge_index, W1a, b1a, g1a, be1a, W1b, b1b, g1b, be1b,
           W2a, b2a, g2a, be2a, W2b, b2b, g2b, be2b):
    n, d = x.shape
    e = edge_index.shape[1]
    h_dim = W1a.shape[0]
    c_dim = W2b.shape[0]

    t_chunks = -(-e // (NS * CHUNK))
    t_chunks += t_chunks % 2
    k0, k1 = 122, 36
    assert NS * (k0 + k1) * CHUNK >= e
    e_pad = NS * (k0 + k1) * CHUNK
    src = edge_index[0]
    dst = edge_index[1]
    if e_pad != e:
        pad = e_pad - e
        src = jnp.concatenate([src, jnp.zeros((pad,), jnp.int32)])
        dst = jnp.concatenate([dst, jnp.full((pad,), n, jnp.int32)])
    src = src.reshape(-1, 1, CHUNK)
    dst = dst.reshape(-1, 1, CHUNK)

    seg = _seg_sum_call(n, d, k0, k1)
    r = lambda a: a.reshape(1, -1)

    part1 = seg(x, src, dst)
    h = _mlp_call(n, h_dim, True)(
        part1, x, W1a, r(b1a), r(g1a), r(be1a), W1b, r(b1b), r(g1b), r(be1b)
    )
    part2 = seg(h, src, dst)
    out = _mlp_call(n, c_dim, False)(
        part2, h, W2a, r(b2a), r(g2a), r(be2a), W2b, r(b2b), r(g2b), r(be2b)
    )
    return out


# speedup vs baseline: 1.0954x; 1.0347x over previous
import functools

import jax
import jax.numpy as jnp
from jax import lax
from jax.experimental import pallas as pl
from jax.experimental.pallas import tpu as pltpu
from jax.experimental.pallas import tpu_sc as plsc

NC = 2
NS = 16
NW = NC * NS
CHUNK = 128


def _seg_sum_call(n, d, k0, k1):
    rows_per_sub = (n // NS) // 8 * 8
    tail = n - rows_per_sub * NS
    n_pad = n + 8
    mesh = plsc.VectorSubcoreMesh(
        core_axis_name="c", subcore_axis_name="s", num_cores=NC, num_subcores=NS
    )

    assert k0 % 2 == 0 and k1 % 2 == 0 and k0 >= 2 and k1 >= 2
    k_max = max(k0, k1)

    @functools.partial(
        pl.kernel,
        out_type=jax.ShapeDtypeStruct((NC, n, d), jnp.float32),
        mesh=mesh,
        scratch_types=[
            pltpu.VMEM_SHARED((n_pad, d), jnp.float32),
            [pltpu.VMEM((1, CHUNK), jnp.int32) for _ in range(2)],
            [pltpu.VMEM((1, CHUNK), jnp.int32) for _ in range(2)],
            [pltpu.VMEM((CHUNK, d), jnp.float32) for _ in range(2)],
            [pltpu.SemaphoreType.DMA for _ in range(2)],
            [pltpu.SemaphoreType.DMA for _ in range(2)],
        ],
    )
    def seg(x_hbm, src_hbm, dst_hbm, out_hbm, acc, sidx, didx, rows, gsem,
            isem):
        c = lax.axis_index("c")
        s = lax.axis_index("s")
        nc = jnp.where(c == 0, k0, k1)
        r0 = s * rows_per_sub
        pltpu.sync_copy(
            x_hbm.at[pl.ds(r0, rows_per_sub)], acc.at[pl.ds(r0, rows_per_sub)]
        )
        if tail:
            @pl.when(s == NS - 1)
            def _():
                t0 = rows_per_sub * NS
                pltpu.sync_copy(
                    x_hbm.at[pl.ds(t0, tail)], acc.at[pl.ds(t0, tail)]
                )
        plsc.subcore_barrier()
        base = jnp.where(c == 0, s * k0, NS * k0 + s * k1)

        def load_idx(k, b):
            pltpu.async_copy(src_hbm.at[k], sidx[b], isem[b])
            pltpu.async_copy(dst_hbm.at[k], didx[b], isem[b])

        def wait_idx(b):
            pltpu.make_async_copy(dst_hbm.at[0], didx[b], isem[b]).wait()
            pltpu.make_async_copy(src_hbm.at[0], sidx[b], isem[b]).wait()

        load_idx(base, 0)
        wait_idx(0)
        pltpu.async_copy(x_hbm.at[sidx[0].at[0]], rows[0], gsem[0])
        load_idx(base + 1, 1)

        def step(k, b):
            @pl.when(k + 1 < nc)
            def _():
                wait_idx(1 - b)
                pltpu.async_copy(x_hbm.at[sidx[1 - b].at[0]], rows[1 - b],
                                 gsem[1 - b])

            @pl.when(k < nc)
            def _():
                pltpu.make_async_copy(
                    x_hbm.at[sidx[b].at[0]], rows[b], gsem[b]).wait()
                pltpu.sync_copy(rows[b], acc.at[didx[b].at[0]], add=True)

            @pl.when(k + 2 < nc)
            def _():
                load_idx(base + k + 2, b)

        def body(j, carry):
            step(j * 2, 0)
            step(j * 2 + 1, 1)
            return carry

        lax.fori_loop(0, k_max // 2, body, 0)
        plsc.subcore_barrier()
        pltpu.sync_copy(
            acc.at[pl.ds(r0, rows_per_sub)],
            out_hbm.at[c].at[pl.ds(r0, rows_per_sub)],
        )
        if tail:
            @pl.when(s == NS - 1)
            def _():
                t0 = rows_per_sub * NS
                pltpu.sync_copy(
                    acc.at[pl.ds(t0, tail)], out_hbm.at[c].at[pl.ds(t0, tail)]
                )

    return seg


def _mlp_body(part, xref, w1, b1, g1, be1, w2, b2, g2, be2, out, *, last_relu):
    h = part[0] + part[1] - xref[...]
    h = lax.dot_general(
        h, w1[...], (((1,), (1,)), ((), ())),
        preferred_element_type=jnp.float32,
    ) + b1[...]
    m = jnp.mean(h, axis=0, keepdims=True)
    v = jnp.mean((h - m) * (h - m), axis=0, keepdims=True)
    h = g1[...] * (h - m) / jnp.sqrt(v + 1e-5) + be1[...]
    h = jnp.maximum(h, 0.0)
    h = lax.dot_general(
        h, w2[...], (((1,), (1,)), ((), ())),
        preferred_element_type=jnp.float32,
    ) + b2[...]
    m = jnp.mean(h, axis=0, keepdims=True)
    v = jnp.mean((h - m) * (h - m), axis=0, keepdims=True)
    h = g2[...] * (h - m) / jnp.sqrt(v + 1e-5) + be2[...]
    if last_relu:
        h = jnp.maximum(h, 0.0)
    out[...] = h


def _mlp_call(n, d_out, last_relu):
    return pl.pallas_call(
        functools.partial(_mlp_body, last_relu=last_relu),
        out_shape=jax.ShapeDtypeStruct((n, d_out), jnp.float32),
    )


def kernel(x, edge_index, W1a, b1a, g1a, be1a, W1b, b1b, g1b, be1b,
           W2a, b2a, g2a, be2a, W2b, b2b, g2b, be2b):
    n, d = x.shape
    e = edge_index.shape[1]
    h_dim = W1a.shape[0]
    c_dim = W2b.shape[0]

    t_chunks = -(-e // (NS * CHUNK))
    t_chunks += t_chunks % 2
    k0, k1 = 128, 30
    assert NS * (k0 + k1) * CHUNK >= e
    e_pad = NS * (k0 + k1) * CHUNK
    src = edge_index[0]
    dst = edge_index[1]
    if e_pad != e:
        pad = e_pad - e
        src = jnp.concatenate([src, jnp.zeros((pad,), jnp.int32)])
        dst = jnp.concatenate([dst, jnp.full((pad,), n, jnp.int32)])
    src = src.reshape(-1, 1, CHUNK)
    dst = dst.reshape(-1, 1, CHUNK)

    seg = _seg_sum_call(n, d, k0, k1)
    r = lambda a: a.reshape(1, -1)

    part1 = seg(x, src, dst)
    h = _mlp_call(n, h_dim, True)(
        part1, x, W1a, r(b1a), r(g1a), r(be1a), W1b, r(b1b), r(g1b), r(be1b)
    )
    part2 = seg(h, src, dst)
    out = _mlp_call(n, c_dim, False)(
        part2, h, W2a, r(b2a), r(g2a), r(be2a), W2b, r(b2b), r(g2b), r(be2b)
    )
    return out


# speedup vs baseline: 1.1097x; 1.0131x over previous
import functools

import jax
import jax.numpy as jnp
from jax import lax
from jax.experimental import pallas as pl
from jax.experimental.pallas import tpu as pltpu
from jax.experimental.pallas import tpu_sc as plsc

NC = 2
NS = 16
NW = NC * NS
CHUNK = 128


def _seg_sum_call(n, d, k0, k1):
    rows_per_sub = (n // NS) // 8 * 8
    tail = n - rows_per_sub * NS
    n_pad = n + 8
    mesh = plsc.VectorSubcoreMesh(
        core_axis_name="c", subcore_axis_name="s", num_cores=NC, num_subcores=NS
    )

    assert k0 % 2 == 0 and k1 % 2 == 0 and k0 >= 2 and k1 >= 2
    k_max = max(k0, k1)

    @functools.partial(
        pl.kernel,
        out_type=jax.ShapeDtypeStruct((NC, n, d), jnp.float32),
        mesh=mesh,
        scratch_types=[
            pltpu.VMEM_SHARED((n_pad, d), jnp.float32),
            [pltpu.VMEM((1, CHUNK), jnp.int32) for _ in range(2)],
            [pltpu.VMEM((1, CHUNK), jnp.int32) for _ in range(2)],
            [pltpu.VMEM((CHUNK, d), jnp.float32) for _ in range(2)],
            [pltpu.SemaphoreType.DMA for _ in range(2)],
            [pltpu.SemaphoreType.DMA for _ in range(2)],
        ],
    )
    def seg(x_hbm, src_hbm, dst_hbm, out_hbm, acc, sidx, didx, rows, gsem,
            isem):
        c = lax.axis_index("c")
        s = lax.axis_index("s")
        nc = jnp.where(c == 0, k0, k1)
        r0 = s * rows_per_sub
        pltpu.sync_copy(
            x_hbm.at[pl.ds(r0, rows_per_sub)], acc.at[pl.ds(r0, rows_per_sub)]
        )
        if tail:
            @pl.when(s == NS - 1)
            def _():
                t0 = rows_per_sub * NS
                pltpu.sync_copy(
                    x_hbm.at[pl.ds(t0, tail)], acc.at[pl.ds(t0, tail)]
                )
        plsc.subcore_barrier()
        base = jnp.where(c == 0, s * k0, NS * k0 + s * k1)

        def load_idx(k, b):
            pltpu.async_copy(src_hbm.at[k], sidx[b], isem[b])
            pltpu.async_copy(dst_hbm.at[k], didx[b], isem[b])

        def wait_idx(b):
            pltpu.make_async_copy(dst_hbm.at[0], didx[b], isem[b]).wait()
            pltpu.make_async_copy(src_hbm.at[0], sidx[b], isem[b]).wait()

        load_idx(base, 0)
        wait_idx(0)
        pltpu.async_copy(x_hbm.at[sidx[0].at[0]], rows[0], gsem[0])
        load_idx(base + 1, 1)

        def step(k, b):
            @pl.when(k + 1 < nc)
            def _():
                wait_idx(1 - b)
                pltpu.async_copy(x_hbm.at[sidx[1 - b].at[0]], rows[1 - b],
                                 gsem[1 - b])

            @pl.when(k < nc)
            def _():
                pltpu.make_async_copy(
                    x_hbm.at[sidx[b].at[0]], rows[b], gsem[b]).wait()
                pltpu.sync_copy(rows[b], acc.at[didx[b].at[0]], add=True)

            @pl.when(k + 2 < nc)
            def _():
                load_idx(base + k + 2, b)

        def body(j, carry):
            step(j * 2, 0)
            step(j * 2 + 1, 1)
            return carry

        lax.fori_loop(0, k_max // 2, body, 0)
        plsc.subcore_barrier()
        pltpu.sync_copy(
            acc.at[pl.ds(r0, rows_per_sub)],
            out_hbm.at[c].at[pl.ds(r0, rows_per_sub)],
        )
        if tail:
            @pl.when(s == NS - 1)
            def _():
                t0 = rows_per_sub * NS
                pltpu.sync_copy(
                    acc.at[pl.ds(t0, tail)], out_hbm.at[c].at[pl.ds(t0, tail)]
                )

    return seg


def _mlp_body(part, xref, w1, b1, g1, be1, w2, b2, g2, be2, out, *, last_relu):
    h = part[0] + part[1] - xref[...]
    h = lax.dot_general(
        h, w1[...], (((1,), (1,)), ((), ())),
        preferred_element_type=jnp.float32,
    ) + b1[...]
    m = jnp.mean(h, axis=0, keepdims=True)
    v = jnp.mean((h - m) * (h - m), axis=0, keepdims=True)
    h = g1[...] * (h - m) / jnp.sqrt(v + 1e-5) + be1[...]
    h = jnp.maximum(h, 0.0)
    h = lax.dot_general(
        h, w2[...], (((1,), (1,)), ((), ())),
        preferred_element_type=jnp.float32,
    ) + b2[...]
    m = jnp.mean(h, axis=0, keepdims=True)
    v = jnp.mean((h - m) * (h - m), axis=0, keepdims=True)
    h = g2[...] * (h - m) / jnp.sqrt(v + 1e-5) + be2[...]
    if last_relu:
        h = jnp.maximum(h, 0.0)
    out[...] = h


def _mlp_call(n, d_out, last_relu):
    return pl.pallas_call(
        functools.partial(_mlp_body, last_relu=last_relu),
        out_shape=jax.ShapeDtypeStruct((n, d_out), jnp.float32),
    )


def kernel(x, edge_index, W1a, b1a, g1a, be1a, W1b, b1b, g1b, be1b,
           W2a, b2a, g2a, be2a, W2b, b2b, g2b, be2b):
    n, d = x.shape
    e = edge_index.shape[1]
    h_dim = W1a.shape[0]
    c_dim = W2b.shape[0]

    t_chunks = -(-e // (NS * CHUNK))
    t_chunks += t_chunks % 2
    k0, k1 = 134, 24
    assert NS * (k0 + k1) * CHUNK >= e
    e_pad = NS * (k0 + k1) * CHUNK
    src = edge_index[0]
    dst = edge_index[1]
    if e_pad != e:
        pad = e_pad - e
        src = jnp.concatenate([src, jnp.zeros((pad,), jnp.int32)])
        dst = jnp.concatenate([dst, jnp.full((pad,), n, jnp.int32)])
    src = src.reshape(-1, 1, CHUNK)
    dst = dst.reshape(-1, 1, CHUNK)

    seg = _seg_sum_call(n, d, k0, k1)
    r = lambda a: a.reshape(1, -1)

    part1 = seg(x, src, dst)
    h = _mlp_call(n, h_dim, True)(
        part1, x, W1a, r(b1a), r(g1a), r(be1a), W1b, r(b1b), r(g1b), r(be1b)
    )
    part2 = seg(h, src, dst)
    out = _mlp_call(n, c_dim, False)(
        part2, h, W2a, r(b2a), r(g2a), r(be2a), W2b, r(b2b), r(g2b), r(be2b)
    )
    return out


# speedup vs baseline: 1.1191x; 1.0085x over previous
import functools

import jax
import jax.numpy as jnp
from jax import lax
from jax.experimental import pallas as pl
from jax.experimental.pallas import tpu as pltpu
from jax.experimental.pallas import tpu_sc as plsc

NC = 2
NS = 16
NW = NC * NS
CHUNK = 128


def _seg_sum_call(n, d, k0, k1):
    rows_per_sub = (n // NS) // 8 * 8
    tail = n - rows_per_sub * NS
    n_pad = n + 8
    mesh = plsc.VectorSubcoreMesh(
        core_axis_name="c", subcore_axis_name="s", num_cores=NC, num_subcores=NS
    )

    assert k0 % 2 == 0 and k1 % 2 == 0 and k0 >= 2 and k1 >= 2
    k_max = max(k0, k1)

    @functools.partial(
        pl.kernel,
        out_type=jax.ShapeDtypeStruct((NC, n, d), jnp.float32),
        mesh=mesh,
        scratch_types=[
            pltpu.VMEM_SHARED((n_pad, d), jnp.float32),
            [pltpu.VMEM((1, CHUNK), jnp.int32) for _ in range(2)],
            [pltpu.VMEM((1, CHUNK), jnp.int32) for _ in range(2)],
            [pltpu.VMEM((CHUNK, d), jnp.float32) for _ in range(2)],
            [pltpu.SemaphoreType.DMA for _ in range(2)],
            [pltpu.SemaphoreType.DMA for _ in range(2)],
        ],
    )
    def seg(x_hbm, src_hbm, dst_hbm, out_hbm, acc, sidx, didx, rows, gsem,
            isem):
        c = lax.axis_index("c")
        s = lax.axis_index("s")
        nc = jnp.where(c == 0, k0, k1)
        r0 = s * rows_per_sub
        pltpu.sync_copy(
            x_hbm.at[pl.ds(r0, rows_per_sub)], acc.at[pl.ds(r0, rows_per_sub)]
        )
        if tail:
            @pl.when(s == NS - 1)
            def _():
                t0 = rows_per_sub * NS
                pltpu.sync_copy(
                    x_hbm.at[pl.ds(t0, tail)], acc.at[pl.ds(t0, tail)]
                )
        plsc.subcore_barrier()
        base = jnp.where(c == 0, s * k0, NS * k0 + s * k1)

        def load_idx(k, b):
            pltpu.async_copy(src_hbm.at[k], sidx[b], isem[b])
            pltpu.async_copy(dst_hbm.at[k], didx[b], isem[b])

        def wait_idx(b):
            pltpu.make_async_copy(dst_hbm.at[0], didx[b], isem[b]).wait()
            pltpu.make_async_copy(src_hbm.at[0], sidx[b], isem[b]).wait()

        load_idx(base, 0)
        wait_idx(0)
        pltpu.async_copy(x_hbm.at[sidx[0].at[0]], rows[0], gsem[0])
        load_idx(base + 1, 1)

        def step(k, b):
            @pl.when(k + 1 < nc)
            def _():
                wait_idx(1 - b)
                pltpu.async_copy(x_hbm.at[sidx[1 - b].at[0]], rows[1 - b],
                                 gsem[1 - b])

            @pl.when(k < nc)
            def _():
                pltpu.make_async_copy(
                    x_hbm.at[sidx[b].at[0]], rows[b], gsem[b]).wait()
                pltpu.sync_copy(rows[b], acc.at[didx[b].at[0]], add=True)

            @pl.when(k + 2 < nc)
            def _():
                load_idx(base + k + 2, b)

        def body(j, carry):
            step(j * 2, 0)
            step(j * 2 + 1, 1)
            return carry

        lax.fori_loop(0, k_max // 2, body, 0)
        plsc.subcore_barrier()
        pltpu.sync_copy(
            acc.at[pl.ds(r0, rows_per_sub)],
            out_hbm.at[c].at[pl.ds(r0, rows_per_sub)],
        )
        if tail:
            @pl.when(s == NS - 1)
            def _():
                t0 = rows_per_sub * NS
                pltpu.sync_copy(
                    acc.at[pl.ds(t0, tail)], out_hbm.at[c].at[pl.ds(t0, tail)]
                )

    return seg


def _mlp_body(part, xref, w1, b1, g1, be1, w2, b2, g2, be2, out, *, last_relu):
    h = part[0] + part[1] - xref[...]
    h = lax.dot_general(
        h, w1[...], (((1,), (1,)), ((), ())),
        preferred_element_type=jnp.float32,
    ) + b1[...]
    m = jnp.mean(h, axis=0, keepdims=True)
    v = jnp.mean((h - m) * (h - m), axis=0, keepdims=True)
    h = g1[...] * (h - m) / jnp.sqrt(v + 1e-5) + be1[...]
    h = jnp.maximum(h, 0.0)
    h = lax.dot_general(
        h, w2[...], (((1,), (1,)), ((), ())),
        preferred_element_type=jnp.float32,
    ) + b2[...]
    m = jnp.mean(h, axis=0, keepdims=True)
    v = jnp.mean((h - m) * (h - m), axis=0, keepdims=True)
    h = g2[...] * (h - m) / jnp.sqrt(v + 1e-5) + be2[...]
    if last_relu:
        h = jnp.maximum(h, 0.0)
    out[...] = h


def _mlp_call(n, d_out, last_relu):
    return pl.pallas_call(
        functools.partial(_mlp_body, last_relu=last_relu),
        out_shape=jax.ShapeDtypeStruct((n, d_out), jnp.float32),
    )


def kernel(x, edge_index, W1a, b1a, g1a, be1a, W1b, b1b, g1b, be1b,
           W2a, b2a, g2a, be2a, W2b, b2b, g2b, be2b):
    n, d = x.shape
    e = edge_index.shape[1]
    h_dim = W1a.shape[0]
    c_dim = W2b.shape[0]

    t_chunks = -(-e // (NS * CHUNK))
    t_chunks += t_chunks % 2
    k0, k1 = 140, 18
    assert NS * (k0 + k1) * CHUNK >= e
    e_pad = NS * (k0 + k1) * CHUNK
    src = edge_index[0]
    dst = edge_index[1]
    if e_pad != e:
        pad = e_pad - e
        src = jnp.concatenate([src, jnp.zeros((pad,), jnp.int32)])
        dst = jnp.concatenate([dst, jnp.full((pad,), n, jnp.int32)])
    src = src.reshape(-1, 1, CHUNK)
    dst = dst.reshape(-1, 1, CHUNK)

    seg = _seg_sum_call(n, d, k0, k1)
    r = lambda a: a.reshape(1, -1)

    part1 = seg(x, src, dst)
    h = _mlp_call(n, h_dim, True)(
        part1, x, W1a, r(b1a), r(g1a), r(be1a), W1b, r(b1b), r(g1b), r(be1b)
    )
    part2 = seg(h, src, dst)
    out = _mlp_call(n, c_dim, False)(
        part2, h, W2a, r(b2a), r(g2a), r(be2a), W2b, r(b2b), r(g2b), r(be2b)
    )
    return out
